# baseline (speedup 1.0000x reference)
import functools

import jax
import jax.numpy as jnp
from jax import lax
from jax.experimental import pallas as pl
from jax.experimental.pallas import tpu as pltpu
from jax.experimental.pallas import tpu_sc as plsc

DIM = 16
NUM_CORES = 2
NUM_SUBCORES = 16
NUM_WORKERS = NUM_CORES * NUM_SUBCORES
CB = 4
CHUNK = 1024
TBLK = 64

_SC_PARAMS = pltpu.CompilerParams(
    use_tc_tiling_on_sc=False, needs_layout_passes=False)


@jax.jit
def _compact_table(table_split):
    nblk = table_split.shape[0]
    per_w = nblk // NUM_WORKERS
    rem = nblk - per_w * NUM_WORKERS
    mesh = plsc.VectorSubcoreMesh(core_axis_name="c", subcore_axis_name="s")

    @functools.partial(
        pl.kernel,
        mesh=mesh,
        out_type=jax.ShapeDtypeStruct(table_split.shape, jnp.float32),
        scratch_types=[pltpu.SemaphoreType.DMA],
        compiler_params=_SC_PARAMS,
    )
    def body(src, dst, sem):
        wid = lax.axis_index("s") * NUM_CORES + lax.axis_index("c")
        b0 = wid * per_w
        pltpu.async_copy(
            src.at[pl.ds(b0, per_w)], dst.at[pl.ds(b0, per_w)], sem).wait()
        if rem:
            def tail():
                t0 = nblk - rem
                pltpu.async_copy(
                    src.at[pl.ds(t0, rem)], dst.at[pl.ds(t0, rem)], sem
                ).wait()
            pl.when(wid == 0)(tail)

    return body(table_split)


@functools.partial(jax.jit, static_argnums=(2, 3))
def _gather_rows(idx_slabs, table_lin, bsz, lsz):
    nbc = bsz // 128
    nlt = lsz // 8
    chunks = nlt * CB
    mesh = plsc.VectorSubcoreMesh(core_axis_name="c", subcore_axis_name="s")

    @functools.partial(
        pl.kernel,
        mesh=mesh,
        out_type=jax.ShapeDtypeStruct((nlt, 8, 2, nbc, 1024), jnp.float32),
        scratch_types=[
            pltpu.VMEM((2, CHUNK), jnp.int32),
            pltpu.VMEM((2, CHUNK, DIM), jnp.float32),
            pltpu.VMEM((2, 8, 2, 1024), jnp.float32),
            pltpu.SemaphoreType.DMA,
            pltpu.SemaphoreType.DMA,
            pltpu.SemaphoreType.DMA,
            pltpu.SemaphoreType.DMA,
            pltpu.SemaphoreType.DMA,
            pltpu.SemaphoreType.DMA,
        ],
        compiler_params=_SC_PARAMS,
    )
    def body(idx_hbm, table_hbm, out_hbm, idx_v, rows_v, tp_v,
             si0, si1, sg0, sg1, so0, so1):
        si = (si0, si1)
        sg = (sg0, sg1)
        so = (so0, so1)
        wid = lax.axis_index("s") * NUM_CORES + lax.axis_index("c")
        col0 = wid * CB
        iota = lax.iota(jnp.int32, 16)
        giota = [g * 16 + iota for g in range(8)]
        dcol = [jnp.full((16,), d, jnp.int32) for d in range(DIM)]
        bsel = [jnp.full((16,), b, jnp.int32) for b in range(2)]

        def idx_copy(c, b):
            return pltpu.make_async_copy(
                idx_hbm.at[c // CB, col0 + c % CB], idx_v.at[b], si[b])

        def gather(b):
            return pltpu.make_async_copy(
                table_hbm.at[idx_v.at[b]], rows_v.at[b], sg[b])

        def store_one(c, b, l8, dt):
            return pltpu.make_async_copy(
                tp_v.at[b, l8, dt],
                out_hbm.at[c // CB, l8, dt, col0 + c % CB], so[b])

        def stores_start(c, b):
            for l8 in range(8):
                for dt in range(2):
                    store_one(c, b, l8, dt).start()

        def stores_wait(c, b):
            for l8 in range(8):
                for dt in range(2):
                    store_one(c, b, l8, dt).wait()

        def transpose(b):
            def per_l8(l8, carry):
                rb = l8 * 128
                for g in range(8):
                    row = rb + giota[g]
                    vs = [plsc.load_gather(rows_v, [bsel[b], row, dcol[d]])
                          for d in range(DIM)]
                    for d in range(DIM):
                        dt, d8 = divmod(d, 8)
                        tp_v[b, l8, dt, pl.ds(d8 * 128 + g * 16, 16)] = vs[d]
                return carry
            lax.fori_loop(0, 8, per_l8, 0)

        idx_copy(0, 0).start()
        idx_copy(0, 0).wait()
        gather(0).start()
        idx_copy(1, 1).start()

        def step(g, carry):
            for b in range(2):
                c = g * 2 + b
                b2 = 1 - b
                def launch_next():
                    idx_copy(c + 1, b2).wait()
                    gather(b2).start()
                pl.when(c + 1 < chunks)(launch_next)
                gather(b).wait()
                pl.when(c + 2 < chunks)(lambda: idx_copy(c + 2, b).start())
                pl.when(c >= 2)(lambda: stores_wait(c, b))
                transpose(b)
                stores_start(c, b)
            return carry

        lax.fori_loop(0, chunks // 2, step, 0)
        for b in range(2):
            stores_wait(0, b)

    return body(idx_slabs, table_lin)


def kernel(X, table):
    b, l = X.shape
    v = table.shape[0]
    idx_slabs = (X.reshape(b // 128, 128, l // 8, 8)
                 .transpose(2, 0, 3, 1).reshape(l // 8, 128, 1024))
    table_lin = _compact_table(table.reshape(v // TBLK, TBLK, DIM))
    p = _gather_rows(idx_slabs, table_lin.reshape(v, DIM), b, l)
    return (p.reshape(l, 2, b // 128, 8, 128)
            .transpose(2, 4, 0, 1, 3).reshape(b, l, DIM))


# speedup vs baseline: 3.5064x; 3.5064x over previous
import functools

import jax
import jax.numpy as jnp
from jax import lax
from jax.experimental import pallas as pl
from jax.experimental.pallas import tpu as pltpu
from jax.experimental.pallas import tpu_sc as plsc

DIM = 16
NUM_CORES = 2
NUM_SUBCORES = 16
NUM_WORKERS = NUM_CORES * NUM_SUBCORES
CB = 4
CHUNK = 1024
TBLK = 64

_SC_PARAMS = pltpu.CompilerParams(
    use_tc_tiling_on_sc=False, needs_layout_passes=False)


@jax.jit
def _compact_table(table_split):
    nblk = table_split.shape[0]
    per_w = nblk // NUM_WORKERS
    rem = nblk - per_w * NUM_WORKERS
    cpb = 61
    groups = per_w // cpb
    gr_rem = per_w - groups * cpb
    mesh = plsc.VectorSubcoreMesh(core_axis_name="c", subcore_axis_name="s")

    @functools.partial(
        pl.kernel,
        mesh=mesh,
        out_type=jax.ShapeDtypeStruct(table_split.shape, jnp.float32),
        scratch_types=[
            pltpu.VMEM((2, cpb, TBLK, DIM), jnp.float32),
            pltpu.SemaphoreType.DMA,
            pltpu.SemaphoreType.DMA,
            pltpu.SemaphoreType.DMA,
            pltpu.SemaphoreType.DMA,
        ],
        compiler_params=_SC_PARAMS,
    )
    def body(src, dst, buf, sin0, sin1, sout0, sout1):
        sin = (sin0, sin1)
        sout = (sout0, sout1)
        wid = lax.axis_index("s") * NUM_CORES + lax.axis_index("c")
        b0 = wid * per_w

        def cin(g, b):
            return pltpu.make_async_copy(
                src.at[pl.ds(b0 + g * cpb, cpb)], buf.at[b], sin[b])

        def cout(g, b):
            return pltpu.make_async_copy(
                buf.at[b], dst.at[pl.ds(b0 + g * cpb, cpb)], sout[b])

        cin(0, 0).start()
        pl.when(groups > 1)(lambda: cin(1, 1).start())

        def step(g, carry):
            for b in range(2):
                i = g * 2 + b

                def run():
                    cin(i, b).wait()
                    cout(i, b).start()
                    cout(i, b).wait()
                    pl.when(i + 2 < groups)(lambda: cin(i + 2, b).start())
                pl.when(i < groups)(run)
            return carry

        lax.fori_loop(0, (groups + 1) // 2, step, 0)

        def small(t0, n):
            c1 = pltpu.make_async_copy(
                src.at[pl.ds(t0, n)], buf.at[0, pl.ds(0, n)], sin[0])
            c1.start()
            c1.wait()
            c2 = pltpu.make_async_copy(
                buf.at[0, pl.ds(0, n)], dst.at[pl.ds(t0, n)], sout[0])
            c2.start()
            c2.wait()

        if gr_rem:
            small(b0 + groups * cpb, gr_rem)
        if rem:
            pl.when(wid == 0)(lambda: small(nblk - rem, rem))

    return body(table_split)


@functools.partial(jax.jit, static_argnums=(2, 3))
def _gather_rows(idx_slabs, table_lin, bsz, lsz):
    nbc = bsz // 128
    nlt = lsz // 8
    chunks = nlt * CB
    mesh = plsc.VectorSubcoreMesh(core_axis_name="c", subcore_axis_name="s")

    @functools.partial(
        pl.kernel,
        mesh=mesh,
        out_type=jax.ShapeDtypeStruct((nlt, 8, 2, nbc, 1024), jnp.float32),
        scratch_types=[
            pltpu.VMEM((2, CHUNK), jnp.int32),
            pltpu.VMEM((2, CHUNK, DIM), jnp.float32),
            pltpu.VMEM((2, 8, 2, 1024), jnp.float32),
            pltpu.SemaphoreType.DMA,
            pltpu.SemaphoreType.DMA,
            pltpu.SemaphoreType.DMA,
            pltpu.SemaphoreType.DMA,
            pltpu.SemaphoreType.DMA,
            pltpu.SemaphoreType.DMA,
        ],
        compiler_params=_SC_PARAMS,
    )
    def body(idx_hbm, table_hbm, out_hbm, idx_v, rows_v, tp_v,
             si0, si1, sg0, sg1, so0, so1):
        si = (si0, si1)
        sg = (sg0, sg1)
        so = (so0, so1)
        wid = lax.axis_index("s") * NUM_CORES + lax.axis_index("c")
        col0 = wid * CB
        iota = lax.iota(jnp.int32, 16)
        giota = [g * 16 + iota for g in range(8)]
        dcol = [jnp.full((16,), d, jnp.int32) for d in range(DIM)]
        bsel = [jnp.full((16,), b, jnp.int32) for b in range(2)]

        def idx_copy(c, b):
            return pltpu.make_async_copy(
                idx_hbm.at[c // CB, col0 + c % CB], idx_v.at[b], si[b])

        def gather(b):
            return pltpu.make_async_copy(
                table_hbm.at[idx_v.at[b]], rows_v.at[b], sg[b])

        def store_one(c, b, l8, dt):
            return pltpu.make_async_copy(
                tp_v.at[b, l8, dt],
                out_hbm.at[c // CB, l8, dt, col0 + c % CB], so[b])

        def stores_start(c, b):
            for l8 in range(8):
                for dt in range(2):
                    store_one(c, b, l8, dt).start()

        def stores_wait(c, b):
            for l8 in range(8):
                for dt in range(2):
                    store_one(c, b, l8, dt).wait()

        def transpose(b):
            def per_l8(l8, carry):
                rb = l8 * 128
                for g in range(8):
                    row = rb + giota[g]
                    vs = [plsc.load_gather(rows_v, [bsel[b], row, dcol[d]])
                          for d in range(DIM)]
                    for d in range(DIM):
                        dt, d8 = divmod(d, 8)
                        tp_v[b, l8, dt, pl.ds(d8 * 128 + g * 16, 16)] = vs[d]
                return carry
            lax.fori_loop(0, 8, per_l8, 0)

        idx_copy(0, 0).start()
        idx_copy(0, 0).wait()
        gather(0).start()
        idx_copy(1, 1).start()

        def step(g, carry):
            for b in range(2):
                c = g * 2 + b
                b2 = 1 - b
                def launch_next():
                    idx_copy(c + 1, b2).wait()
                    gather(b2).start()
                pl.when(c + 1 < chunks)(launch_next)
                gather(b).wait()
                pl.when(c + 2 < chunks)(lambda: idx_copy(c + 2, b).start())
                pl.when(c >= 2)(lambda: stores_wait(c, b))
                transpose(b)
                stores_start(c, b)
            return carry

        lax.fori_loop(0, chunks // 2, step, 0)
        for b in range(2):
            stores_wait(0, b)

    return body(idx_slabs, table_lin)


def kernel(X, table):
    b, l = X.shape
    v = table.shape[0]
    idx_slabs = (X.reshape(b // 128, 128, l // 8, 8)
                 .transpose(2, 0, 3, 1).reshape(l // 8, 128, 1024))
    table_lin = _compact_table(table.reshape(v // TBLK, TBLK, DIM))
    p = _gather_rows(idx_slabs, table_lin.reshape(v, DIM), b, l)
    return (p.reshape(l, 2, b // 128, 8, 128)
            .transpose(2, 4, 0, 1, 3).reshape(b, l, DIM))
